# Initial kernel scaffold; baseline (speedup 1.0000x reference)
import jax
import jax.numpy as jnp
from jax.experimental import pallas as pl


def kernel(x, edge_index, W1, b1, W2, b2):
    raise NotImplementedError("write your pallas kernel here")


# speedup vs baseline: 10.9071x; 10.9071x over previous
import functools

import jax
import jax.numpy as jnp
from jax import lax
from jax.experimental import pallas as pl
from jax.experimental.pallas import tpu as pltpu
from jax.experimental.pallas import tpu_sc as plsc

_EPS = 0.001

_N = 10000
_D = 128
_E = 320000
_NC = 2
_NS = 16
_NW = _NC * _NS
_EPW = _E // _NW
_C = 80
_NCH = _EPW // _C
_RPB = 640


def _make_agg():
  mesh = plsc.VectorSubcoreMesh(core_axis_name="c", subcore_axis_name="s")

  @functools.partial(
      pl.kernel,
      mesh=mesh,
      out_type=jax.ShapeDtypeStruct((_NC, _N, _D), jnp.float32),
      scratch_types=[
          pltpu.VMEM((_EPW,), jnp.int32),
          pltpu.VMEM((_NCH, _C), jnp.int32),
          pltpu.VMEM((_C, _D), jnp.float32),
          pltpu.VMEM((_C, _D), jnp.float32),
          pltpu.VMEM_SHARED((_N, _D), jnp.float32),
          pltpu.SemaphoreType.DMA,
          pltpu.SemaphoreType.DMA,
      ],
  )
  def agg(x_hbm, src_hbm, dst_hbm, out_hbm,
          srcv, dstv, rows_a, rows_b, agg_sh, sem_a, sem_b):
    cid = lax.axis_index("c")
    sid = lax.axis_index("s")
    wid = cid * _NS + sid

    ebase = pl.multiple_of(wid * _EPW, 8)
    pltpu.sync_copy(src_hbm.at[pl.ds(ebase, _EPW)], srcv)
    pltpu.sync_copy(dst_hbm.at[wid], dstv)

    def src_at(j):
      return srcv.at[pl.ds(pl.multiple_of(j * _C, 8), _C)]

    z = jnp.zeros((16,), jnp.float32)

    def zero_row(i, carry):
      for k in range(_D // 16):
        rows_a[i, pl.ds(k * 16, 16)] = z
      return carry

    lax.fori_loop(0, _C, zero_row, 0)
    zbase = pl.multiple_of(sid * _RPB, 8)

    @pl.when(sid < _NS - 1)
    def _():
      for t in range(_RPB // _C):
        pltpu.sync_copy(rows_a, agg_sh.at[pl.ds(zbase + t * _C, _C)])

    @pl.when(sid == _NS - 1)
    def _():
      for t in range((_N - 15 * _RPB) // _C):
        pltpu.sync_copy(rows_a, agg_sh.at[pl.ds(15 * _RPB + t * _C, _C)])

    plsc.subcore_barrier()

    pltpu.async_copy(x_hbm.at[src_at(0)], rows_a, sem_a)

    def body(jj, carry):
      j0 = 2 * jj
      pltpu.async_copy(x_hbm.at[src_at(j0 + 1)], rows_b, sem_b)
      pltpu.make_async_copy(x_hbm.at[src_at(j0)], rows_a, sem_a).wait()
      pltpu.sync_copy(rows_a, agg_sh.at[dstv.at[j0]], add=True)
      pltpu.async_copy(x_hbm.at[src_at(j0 + 2)], rows_a, sem_a)
      pltpu.make_async_copy(x_hbm.at[src_at(j0 + 1)], rows_b, sem_b).wait()
      pltpu.sync_copy(rows_b, agg_sh.at[dstv.at[j0 + 1]], add=True)
      return carry

    lax.fori_loop(0, (_NCH - 1) // 2, body, 0)
    pltpu.make_async_copy(x_hbm.at[src_at(_NCH - 1)], rows_a, sem_a).wait()
    pltpu.sync_copy(rows_a, agg_sh.at[dstv.at[_NCH - 1]], add=True)

    plsc.subcore_barrier()

    @pl.when(sid < _NS - 1)
    def _():
      pltpu.sync_copy(agg_sh.at[pl.ds(zbase, _RPB)],
                      out_hbm.at[cid, pl.ds(zbase, _RPB)])

    @pl.when(sid == _NS - 1)
    def _():
      pltpu.sync_copy(agg_sh.at[pl.ds(15 * _RPB, _N - 15 * _RPB)],
                      out_hbm.at[cid, pl.ds(15 * _RPB, _N - 15 * _RPB)])

  return agg


_BM = 400


def _mlp_body(x_ref, p0_ref, p1_ref, w1_ref, b1_ref, w2_ref, b2_ref, o_ref):
  h = (1.0 + _EPS) * x_ref[...] + p0_ref[...] + p1_ref[...]
  h = jnp.dot(h, w1_ref[...], preferred_element_type=jnp.float32) + b1_ref[...]
  h = jnp.maximum(h, 0.0)
  o_ref[...] = jnp.dot(h, w2_ref[...], preferred_element_type=jnp.float32) + b2_ref[...]


def _mlp(x, p0, p1, W1, b1, W2, b2):
  return pl.pallas_call(
      _mlp_body,
      grid=(_N // _BM,),
      in_specs=[
          pl.BlockSpec((_BM, _D), lambda i: (i, 0)),
          pl.BlockSpec((_BM, _D), lambda i: (i, 0)),
          pl.BlockSpec((_BM, _D), lambda i: (i, 0)),
          pl.BlockSpec((_D, _D), lambda i: (0, 0)),
          pl.BlockSpec((1, _D), lambda i: (0, 0)),
          pl.BlockSpec((_D, _D), lambda i: (0, 0)),
          pl.BlockSpec((1, _D), lambda i: (0, 0)),
      ],
      out_specs=pl.BlockSpec((_BM, _D), lambda i: (i, 0)),
      out_shape=jax.ShapeDtypeStruct((_N, _D), jnp.float32),
  )(x, p0, p1, W1, b1.reshape(1, _D), W2, b2.reshape(1, _D))


def kernel(x, edge_index, W1, b1, W2, b2):
  src = edge_index[0].astype(jnp.int32)
  dst = edge_index[1].astype(jnp.int32).reshape(_NW, _NCH, _C)
  parts = _make_agg()(x, src, dst)
  return _mlp(x, parts[0], parts[1], W1, b1, W2, b2)
